# baseline (speedup 1.0000x reference)
import functools

import jax
import jax.numpy as jnp
from jax import lax
from jax.experimental import pallas as pl
from jax.experimental.pallas import tpu as pltpu
from jax.experimental.pallas import tpu_sc as plsc

_CHUNK_ROWS = 32
_NBUF = 3
_BS = 1024


@functools.cache
def _sc_copy(b, s, d, dtype):
    info = plsc.get_sparse_core_info()
    nw = info.num_cores * info.num_subcores
    rows_per_w = s // nw
    n_chunks = rows_per_w // _CHUNK_ROWS
    mesh = plsc.VectorSubcoreMesh(core_axis_name="c", subcore_axis_name="s")

    @functools.partial(
        pl.kernel,
        mesh=mesh,
        out_type=jax.ShapeDtypeStruct((b, s, d), dtype),
        scratch_types=[
            pltpu.VMEM((_NBUF, _CHUNK_ROWS, d), dtype),
            pltpu.SemaphoreType.DMA,
            pltpu.SemaphoreType.DMA,
        ],
    )
    def k(emb_hbm, out_hbm, buf, rsem, wsem):
        wid = lax.axis_index("s") * info.num_cores + lax.axis_index("c")
        base = wid * rows_per_w

        def rd(c):
            off = base + c * _CHUNK_ROWS
            return pltpu.async_copy(
                emb_hbm.at[pl.ds(off, _CHUNK_ROWS), :], buf.at[c % _NBUF], rsem
            )

        def wr(c):
            off = base + c * _CHUNK_ROWS
            return [
                pltpu.async_copy(
                    buf.at[c % _NBUF], out_hbm.at[bi, pl.ds(off, _CHUNK_ROWS), :], wsem
                )
                for bi in range(b)
            ]

        reads = {}
        writes = {}
        for c in range(min(2, n_chunks)):
            reads[c] = rd(c)
        for c in range(n_chunks):
            reads[c].wait()
            writes[c] = wr(c)
            n = c + 2
            if n < n_chunks:
                prev = n - _NBUF
                if prev >= 0:
                    for w in writes[prev]:
                        w.wait()
                    del writes[prev]
                reads[n] = rd(n)
        for c in sorted(writes):
            for w in writes[c]:
                w.wait()

    return k


def _tc_body(emb_ref, out_ref):
    out_ref[...] = jnp.broadcast_to(emb_ref[...][None], out_ref.shape)


def _tc_copy(b, s, d, dtype, emb):
    return pl.pallas_call(
        _tc_body,
        grid=(s // _BS,),
        in_specs=[pl.BlockSpec((_BS, d), lambda i: (i, 0))],
        out_specs=pl.BlockSpec((b, _BS, d), lambda i: (0, i, 0)),
        out_shape=jax.ShapeDtypeStruct((b, s, d), dtype),
    )(emb)


def kernel(x, emb):
    b, s, d = x.shape
    sc_out = _sc_copy(1, s, d, emb.dtype)(emb)
    tc_out = _tc_copy(b - 1, s, d, emb.dtype, emb)
    return jnp.concatenate([tc_out, sc_out], axis=0)


# speedup vs baseline: 3.5981x; 3.5981x over previous
import functools

import jax
import jax.numpy as jnp
from jax.experimental import pallas as pl
from jax.experimental.pallas import tpu as pltpu

_BS = 1024


def _body(emb_hbm, out_hbm, buf, rsem, wsem, *, b, s, d):
    n = s // _BS
    reads = []
    for c in range(n):
        cp = pltpu.make_async_copy(
            emb_hbm.at[pl.ds(c * _BS, _BS), :], buf.at[c], rsem.at[c]
        )
        cp.start()
        reads.append(cp)
    writes = []
    for c in range(n):
        reads[c].wait()
        for bi in range(b):
            w = pltpu.make_async_copy(
                buf.at[c], out_hbm.at[bi, pl.ds(c * _BS, _BS), :], wsem
            )
            w.start()
            writes.append(w)
    for w in writes:
        w.wait()


def kernel(x, emb):
    b, s, d = x.shape
    n = s // _BS
    return pl.pallas_call(
        functools.partial(_body, b=b, s=s, d=d),
        in_specs=[pl.BlockSpec(memory_space=pl.ANY)],
        out_specs=pl.BlockSpec(memory_space=pl.ANY),
        out_shape=jax.ShapeDtypeStruct((b, s, d), emb.dtype),
        scratch_shapes=[
            pltpu.VMEM((n, _BS, d), emb.dtype),
            pltpu.SemaphoreType.DMA((n,)),
            pltpu.SemaphoreType.DMA,
        ],
    )(emb)
